# baseline (speedup 1.0000x reference)
import jax
import jax.numpy as jnp
from jax import lax
from jax.experimental import pallas as pl
from jax.experimental.pallas import tpu as pltpu
from jax.experimental.pallas import tpu_sc as plsc

BATCH = 16384
FACTORS = 64
L = 16
NC, NS = 2, 16
NW = NC * NS
BPW = BATCH // NW
NG = BPW // L
GU = 16


def _body(user_r, item_i_r, item_j_r, uf3_r, if3_r, out_r,
          idx_u, idx_i, idx_j, bu, bi, bj, tbuf, out_v, sem):
    wid = lax.axis_index("s") * NC + lax.axis_index("c")

    pltpu.sync_copy(user_r.at[wid], idx_u)
    pltpu.sync_copy(item_i_r.at[wid], idx_i)
    pltpu.sync_copy(item_j_r.at[wid], idx_j)

    def group(g, carry):
        kuv = idx_u[pl.ds(g * L, L)]
        kiv = idx_i[pl.ds(g * L, L)]
        kjv = idx_j[pl.ds(g * L, L)]
        guv = lax.shift_right_logical(kuv, 4)
        giv = lax.shift_right_logical(kiv, 4)
        gjv = lax.shift_right_logical(kjv, 4)
        suv = lax.bitwise_and(kuv, jnp.int32(GU - 1))
        siv = lax.bitwise_and(kiv, jnp.int32(GU - 1))
        sjv = lax.bitwise_and(kjv, jnp.int32(GU - 1))
        cu = pltpu.async_copy(uf3_r.at[guv], bu, sem)
        ci = pltpu.async_copy(if3_r.at[giv], bi, sem)
        cj = pltpu.async_copy(if3_r.at[gjv], bj, sem)
        cu.wait()
        ci.wait()
        cj.wait()
        lanes = lax.iota(jnp.int32, L)
        for s in range(L):
            su = suv[s]
            si = siv[s]
            sj = sjv[s]
            ru_ = lax.shift_right_logical(su, 1)
            ri_ = lax.shift_right_logical(si, 1)
            rj_ = lax.shift_right_logical(sj, 1)
            hu = lax.bitwise_and(su, jnp.int32(1)) * (FACTORS)
            hi = lax.bitwise_and(si, jnp.int32(1)) * (FACTORS)
            hj = lax.bitwise_and(sj, jnp.int32(1)) * (FACTORS)
            acc = jnp.zeros((L,), jnp.float32)
            for c in range(FACTORS // L):
                u = bu[s, ru_, pl.ds(hu + c * L, L)]
                vi = bi[s, ri_, pl.ds(hi + c * L, L)]
                vj = bj[s, rj_, pl.ds(hj + c * L, L)]
                acc = acc + u * (vi - vj)
            plsc.store_scatter(
                tbuf, [lanes, jnp.full((L,), s, jnp.int32)], acc)
        tot = tbuf[0, :]
        for r in range(1, L):
            tot = tot + tbuf[r, :]
        out_v[pl.ds(g * L, L)] = tot
        return carry

    lax.fori_loop(0, NG, group, 0)
    pltpu.sync_copy(out_v, out_r.at[pl.ds(wid * BPW, BPW)])


def kernel(user, item_i, item_j, user_factors, item_factors):
    user2 = user.reshape(NW, BPW)
    ii2 = item_i.reshape(NW, BPW)
    ij2 = item_j.reshape(NW, BPW)
    uf3 = user_factors.reshape(62500, 8, 2 * FACTORS)
    if3 = item_factors.reshape(62500, 8, 2 * FACTORS)
    mesh = plsc.VectorSubcoreMesh(core_axis_name="c", subcore_axis_name="s")
    k = pl.kernel(
        _body,
        out_type=jax.ShapeDtypeStruct((BATCH,), jnp.float32),
        mesh=mesh,
        compiler_params=pltpu.CompilerParams(needs_layout_passes=False),
        scratch_types=[
            pltpu.VMEM((BPW,), jnp.int32),
            pltpu.VMEM((BPW,), jnp.int32),
            pltpu.VMEM((BPW,), jnp.int32),
            pltpu.VMEM((L, 8, 2 * FACTORS), jnp.float32),
            pltpu.VMEM((L, 8, 2 * FACTORS), jnp.float32),
            pltpu.VMEM((L, 8, 2 * FACTORS), jnp.float32),
            pltpu.VMEM((L, L), jnp.float32),
            pltpu.VMEM((BPW,), jnp.float32),
            pltpu.SemaphoreType.DMA,
        ],
    )
    return k(user2, ii2, ij2, uf3, if3)
